# Initial kernel scaffold; baseline (speedup 1.0000x reference)
import jax
import jax.numpy as jnp
from jax.experimental import pallas as pl


def kernel(x, edge_index, W1l, b1l, W1r, b1r, W2l, b2l, W2r, b2r, W3l, b3l, W3r, b3r, Wp, bp, Wo, bo):
    raise NotImplementedError("write your pallas kernel here")


# speedup vs baseline: 2.8654x; 2.8654x over previous
import functools

import jax
import jax.numpy as jnp
from jax import lax
from jax.experimental import pallas as pl
from jax.experimental.pallas import tpu as pltpu
from jax.experimental.pallas import tpu_sc as plsc

NUM_SC = 2
NUM_SUBCORES = 16
BATCH = 128
ACC_ROWS_PER_SUBCORE = 640


def _sc_agg(nch, n, e_pad, with_counts):
    assert nch % NUM_SC == 0
    nch_per_core = nch // NUM_SC
    per_sub = e_pad // NUM_SUBCORES
    nb = per_sub // BATCH
    assert nb * BATCH == per_sub
    n_acc = ACC_ROWS_PER_SUBCORE * NUM_SUBCORES
    assert n < n_acc
    rows_out = ((n // NUM_SUBCORES + 7) // 8) * 8
    rows_last = n - rows_out * (NUM_SUBCORES - 1)
    assert 0 < rows_last <= rows_out

    mesh = plsc.VectorSubcoreMesh(core_axis_name="c", subcore_axis_name="s")
    out_type = [jax.ShapeDtypeStruct((n, 128), jnp.float32) for _ in range(nch)]
    if with_counts:
        out_type.append(jax.ShapeDtypeStruct((n, 128), jnp.float32))

    scratch = [
        pltpu.VMEM((BATCH,), jnp.int32),
        pltpu.VMEM((BATCH,), jnp.int32),
        pltpu.VMEM((BATCH, 128), jnp.float32),
        pltpu.VMEM_SHARED((n_acc, 128), jnp.float32),
        pltpu.SemaphoreType.DMA,
    ]

    @functools.partial(pl.kernel, mesh=mesh, out_type=out_type,
                       scratch_types=scratch)
    def agg_kernel(*refs):
        data = refs[:nch]
        src_hbm, dst_hbm, zeros_hbm, ones_hbm = refs[nch:nch + 4]
        outs = refs[nch + 4:nch + 4 + nch]
        cnt_out = refs[nch + 4 + nch] if with_counts else None
        src_v, dst_v, rows_v, acc, sem = refs[-5:]

        core = lax.axis_index("c")
        sub = lax.axis_index("s")
        e0 = sub * per_sub

        def run_chunk(data_ref, out_ref, flush_core0_only=False):
            for z in range(ACC_ROWS_PER_SUBCORE // BATCH):
                pltpu.sync_copy(
                    zeros_hbm,
                    acc.at[pl.ds(sub * ACC_ROWS_PER_SUBCORE + z * BATCH, BATCH)])
            if data_ref is None:
                pltpu.sync_copy(ones_hbm, rows_v)
            plsc.subcore_barrier()

            @pl.loop(0, nb)
            def _(b):
                base = e0 + b * BATCH
                pltpu.sync_copy(dst_hbm.at[pl.ds(base, BATCH)], dst_v)
                if data_ref is not None:
                    pltpu.sync_copy(src_hbm.at[pl.ds(base, BATCH)], src_v)
                    pltpu.async_copy(data_ref.at[src_v], rows_v, sem).wait()
                pltpu.sync_copy(rows_v, acc.at[dst_v], add=True)

            plsc.subcore_barrier()

            fl = (core == 0) if flush_core0_only else (core >= 0)

            @pl.when(fl & (sub < NUM_SUBCORES - 1))
            def _():
                pltpu.sync_copy(acc.at[pl.ds(sub * rows_out, rows_out)],
                                out_ref.at[pl.ds(sub * rows_out, rows_out)])

            @pl.when(fl & (sub == NUM_SUBCORES - 1))
            def _():
                pltpu.sync_copy(acc.at[pl.ds(sub * rows_out, rows_last)],
                                out_ref.at[pl.ds(sub * rows_out, rows_last)])

            plsc.subcore_barrier()

        for j in range(nch_per_core):
            for c in range(NUM_SC):
                @pl.when(core == c)
                def _(j=j, c=c):
                    run_chunk(data[j * NUM_SC + c], outs[j * NUM_SC + c])

        if with_counts:
            run_chunk(None, cnt_out, flush_core0_only=True)

    return agg_kernel


def _selfmm_dense(x, w, b):
    n, fin = x.shape
    h = w.shape[1]
    blk = 1000

    def body(x_ref, w_ref, b_ref, o_ref):
        o_ref[...] = (jnp.dot(x_ref[...], w_ref[...],
                              preferred_element_type=jnp.float32)
                      + b_ref[...])

    return pl.pallas_call(
        body,
        grid=(n // blk,),
        in_specs=[
            pl.BlockSpec((blk, fin), lambda i: (i, 0)),
            pl.BlockSpec((fin, h), lambda i: (0, 0)),
            pl.BlockSpec((1, h), lambda i: (0, 0)),
        ],
        out_specs=pl.BlockSpec((blk, h), lambda i: (i, 0)),
        out_shape=jax.ShapeDtypeStruct((n, h), jnp.float32),
    )(x, w, b.reshape(1, h))


def _selfmm_chunks(chunks, w, b):
    k = len(chunks)
    n = chunks[0].shape[0]
    h = w.shape[1]
    blk = 1000

    def body(*refs):
        cs = refs[:k]
        w_ref, b_ref, o_ref = refs[k], refs[k + 1], refs[k + 2]
        acc = jnp.broadcast_to(b_ref[...], (blk, h))
        for i in range(k):
            acc = acc + jnp.dot(cs[i][...], w_ref[i * 128:(i + 1) * 128, :],
                                preferred_element_type=jnp.float32)
        o_ref[...] = acc

    return pl.pallas_call(
        body,
        grid=(n // blk,),
        in_specs=[pl.BlockSpec((blk, 128), lambda i: (i, 0))] * k + [
            pl.BlockSpec((k * 128, h), lambda i: (0, 0)),
            pl.BlockSpec((1, h), lambda i: (0, 0)),
        ],
        out_specs=pl.BlockSpec((blk, h), lambda i: (i, 0)),
        out_shape=jax.ShapeDtypeStruct((n, h), jnp.float32),
    )(*chunks, w, b.reshape(1, h))


def _combine(agg_chunks, cnt, s_term, wl):
    k = len(agg_chunks)
    n = agg_chunks[0].shape[0]
    h = wl.shape[1]
    ko = h // 128
    blk = 1000

    def body(*refs):
        aggs = refs[:k]
        cnt_ref, s_ref, w_ref = refs[k], refs[k + 1], refs[k + 2]
        outs = refs[k + 3:]
        acc = jnp.zeros((blk, h), jnp.float32)
        for i in range(k):
            acc = acc + jnp.dot(aggs[i][...], w_ref[i * 128:(i + 1) * 128, :],
                                preferred_element_type=jnp.float32)
        inv = 1.0 / jnp.maximum(cnt_ref[...][:, :1], 1.0)
        hv = jnp.maximum(acc * inv + s_ref[...], 0.0)
        for i in range(ko):
            outs[i][...] = hv[:, i * 128:(i + 1) * 128]

    return pl.pallas_call(
        body,
        grid=(n // blk,),
        in_specs=[pl.BlockSpec((blk, 128), lambda i: (i, 0))] * k + [
            pl.BlockSpec((blk, 128), lambda i: (i, 0)),
            pl.BlockSpec((blk, h), lambda i: (i, 0)),
            pl.BlockSpec((k * 128, h), lambda i: (0, 0)),
        ],
        out_specs=[pl.BlockSpec((blk, 128), lambda i: (i, 0))] * ko,
        out_shape=[jax.ShapeDtypeStruct((n, 128), jnp.float32)] * ko,
    )(*agg_chunks, cnt, s_term, wl)


def _final(agg_chunks, cnt, s_term, w3l, wp, bp, wo, bo):
    k = len(agg_chunks)
    n = agg_chunks[0].shape[0]
    h = w3l.shape[1]
    c = wo.shape[1]
    blk = 1000

    def body(*refs):
        aggs = refs[:k]
        (cnt_ref, s_ref, w3l_ref, wp_ref, bp_ref, wo_ref, bo_ref,
         o_ref) = refs[k:]
        acc = jnp.zeros((blk, h), jnp.float32)
        for i in range(k):
            acc = acc + jnp.dot(aggs[i][...], w3l_ref[i * 128:(i + 1) * 128, :],
                                preferred_element_type=jnp.float32)
        inv = 1.0 / jnp.maximum(cnt_ref[...][:, :1], 1.0)
        h3 = acc * inv + s_ref[...]
        h4 = jnp.maximum(jnp.dot(h3, wp_ref[...],
                                 preferred_element_type=jnp.float32)
                         + bp_ref[...], 0.0)
        o_ref[...] = (jnp.dot(h4, wo_ref[...],
                              preferred_element_type=jnp.float32)
                      + bo_ref[...])

    return pl.pallas_call(
        body,
        grid=(n // blk,),
        in_specs=[pl.BlockSpec((blk, 128), lambda i: (i, 0))] * k + [
            pl.BlockSpec((blk, 128), lambda i: (i, 0)),
            pl.BlockSpec((blk, h), lambda i: (i, 0)),
            pl.BlockSpec((k * 128, h), lambda i: (0, 0)),
            pl.BlockSpec((h, h), lambda i: (0, 0)),
            pl.BlockSpec((1, h), lambda i: (0, 0)),
            pl.BlockSpec((h, c), lambda i: (0, 0)),
            pl.BlockSpec((1, c), lambda i: (0, 0)),
        ],
        out_specs=pl.BlockSpec((blk, c), lambda i: (i, 0)),
        out_shape=jax.ShapeDtypeStruct((n, c), jnp.float32),
    )(*agg_chunks, cnt, s_term, w3l, wp, bp.reshape(1, h), wo,
      bo.reshape(1, c))


def kernel(x, edge_index, W1l, b1l, W1r, b1r, W2l, b2l, W2r, b2r,
           W3l, b3l, W3r, b3r, Wp, bp, Wo, bo):
    n, f_in = x.shape
    e = edge_index.shape[1]
    src = edge_index[0]
    dst = edge_index[1]

    grain = NUM_SUBCORES * BATCH
    e_pad = ((e + grain - 1) // grain) * grain
    if e_pad != e:
        src = jnp.concatenate([src, jnp.zeros((e_pad - e,), jnp.int32)])
        dst = jnp.concatenate([dst, jnp.full((e_pad - e,), n, jnp.int32)])

    zeros128 = jnp.zeros((BATCH, 128), jnp.float32)
    ones128 = jnp.ones((BATCH, 128), jnp.float32)

    k_in = f_in // 128
    x_chunks = [x[:, i * 128:(i + 1) * 128] for i in range(k_in)]

    agg1 = _sc_agg(k_in, n, e_pad, True)(
        *x_chunks, src, dst, zeros128, ones128)
    cnt = agg1[-1]
    s1 = _selfmm_dense(x, W1r, b1l + b1r)
    h1 = _combine(list(agg1[:k_in]), cnt, s1, W1l)

    k_h = len(h1)
    sc_h = _sc_agg(k_h, n, e_pad, False)
    agg2 = sc_h(*h1, src, dst, zeros128, ones128)
    s2 = _selfmm_chunks(h1, W2r, b2l + b2r)
    h2 = _combine(list(agg2), cnt, s2, W2l)

    agg3 = sc_h(*h2, src, dst, zeros128, ones128)
    s3 = _selfmm_chunks(h2, W3r, b3l + b3r)
    return _final(list(agg3), cnt, s3, W3l, Wp, bp, Wo, bo)


# speedup vs baseline: 3.1696x; 1.1062x over previous
import functools

import jax
import jax.numpy as jnp
from jax import lax
from jax.experimental import pallas as pl
from jax.experimental.pallas import tpu as pltpu
from jax.experimental.pallas import tpu_sc as plsc

NUM_SC = 2
NUM_SUBCORES = 16
BATCH = 128
PAIR = 2
NBLK = 2
ACC_ROWS_PER_SUBCORE = 640


def _sc_agg(nch, n, e_pad, with_counts):
    assert nch % NUM_SC == 0
    nch_per_core = nch // NUM_SC
    per_sub = e_pad // NUM_SUBCORES
    nb = per_sub // BATCH
    assert nb * BATCH == per_sub
    n_acc = ACC_ROWS_PER_SUBCORE * NUM_SUBCORES
    assert n < n_acc
    rows_out = ((n // NUM_SUBCORES + 7) // 8) * 8
    rows_last = n - rows_out * (NUM_SUBCORES - 1)
    assert 0 < rows_last <= rows_out

    assert nb % (PAIR * NBLK) == 0
    nb_blk = nb // NBLK
    per_blk = nb_blk * BATCH
    np2 = nb_blk // PAIR

    mesh = plsc.VectorSubcoreMesh(core_axis_name="c", subcore_axis_name="s")
    out_type = [jax.ShapeDtypeStruct((n, 128), jnp.float32) for _ in range(nch)]
    if with_counts:
        out_type.append(jax.ShapeDtypeStruct((n, 128), jnp.float32))

    scratch = [
        pltpu.VMEM((per_blk,), jnp.int32),
        pltpu.VMEM((per_blk,), jnp.int32),
        pltpu.VMEM((BATCH, 128), jnp.float32),
        pltpu.VMEM((BATCH, 128), jnp.float32),
        pltpu.VMEM_SHARED((n_acc, 128), jnp.float32),
        pltpu.SemaphoreType.DMA,
        pltpu.SemaphoreType.DMA,
    ]

    @functools.partial(pl.kernel, mesh=mesh, out_type=out_type,
                       scratch_types=scratch)
    def agg_kernel(*refs):
        data = refs[:nch]
        src_hbm, dst_hbm, zeros_hbm, ones_hbm = refs[nch:nch + 4]
        outs = refs[nch + 4:nch + 4 + nch]
        cnt_out = refs[nch + 4 + nch] if with_counts else None
        src_blk, dst_blk, rows_a, rows_b, acc, sem_a, sem_b = refs[-7:]

        core = lax.axis_index("c")
        sub = lax.axis_index("s")
        e0 = sub * per_sub

        def src_ix(b):
            return src_blk.at[pl.ds(b * BATCH, BATCH)]

        def dst_ix(b):
            return dst_blk.at[pl.ds(b * BATCH, BATCH)]

        def run_chunk(data_ref, out_ref, flush_core0_only=False):
            zh = [pltpu.async_copy(
                zeros_hbm,
                acc.at[pl.ds(sub * ACC_ROWS_PER_SUBCORE + z * BATCH, BATCH)],
                sem_b)
                for z in range(ACC_ROWS_PER_SUBCORE // BATCH)]
            for h in zh:
                h.wait()
            if data_ref is None:
                pltpu.sync_copy(ones_hbm, rows_a)
            plsc.subcore_barrier()

            for blk in range(NBLK):
                base = e0 + blk * per_blk
                pltpu.sync_copy(dst_hbm.at[pl.ds(base, per_blk)], dst_blk)
                if data_ref is None:
                    @pl.loop(0, nb_blk)
                    def _(b):
                        pltpu.sync_copy(rows_a, acc.at[dst_ix(b)], add=True)
                else:
                    pltpu.sync_copy(src_hbm.at[pl.ds(base, per_blk)], src_blk)
                    pltpu.async_copy(data_ref.at[src_ix(0)], rows_a, sem_a)

                    @pl.loop(0, np2)
                    def _(i):
                        b0 = i * PAIR
                        pltpu.async_copy(data_ref.at[src_ix(b0 + 1)], rows_b,
                                         sem_b)
                        pltpu.make_async_copy(data_ref.at[src_ix(b0)], rows_a,
                                              sem_a).wait()
                        pltpu.sync_copy(rows_a, acc.at[dst_ix(b0)], add=True)

                        @pl.when(i < np2 - 1)
                        def _():
                            pltpu.async_copy(data_ref.at[src_ix(b0 + PAIR)],
                                             rows_a, sem_a)

                        pltpu.make_async_copy(data_ref.at[src_ix(b0 + 1)],
                                              rows_b, sem_b).wait()
                        pltpu.sync_copy(rows_b, acc.at[dst_ix(b0 + 1)],
                                        add=True)

            plsc.subcore_barrier()

            fl = (core == 0) if flush_core0_only else (core >= 0)

            @pl.when(fl & (sub < NUM_SUBCORES - 1))
            def _():
                pltpu.sync_copy(acc.at[pl.ds(sub * rows_out, rows_out)],
                                out_ref.at[pl.ds(sub * rows_out, rows_out)])

            @pl.when(fl & (sub == NUM_SUBCORES - 1))
            def _():
                pltpu.sync_copy(acc.at[pl.ds(sub * rows_out, rows_last)],
                                out_ref.at[pl.ds(sub * rows_out, rows_last)])

            plsc.subcore_barrier()

        for j in range(nch_per_core):
            for c in range(NUM_SC):
                @pl.when(core == c)
                def _(j=j, c=c):
                    run_chunk(data[j * NUM_SC + c], outs[j * NUM_SC + c])

        if with_counts:
            run_chunk(None, cnt_out, flush_core0_only=True)

    return agg_kernel


def _selfmm_dense(x, w, b):
    n, fin = x.shape
    h = w.shape[1]
    blk = 1000

    def body(x_ref, w_ref, b_ref, o_ref):
        o_ref[...] = (jnp.dot(x_ref[...], w_ref[...],
                              preferred_element_type=jnp.float32)
                      + b_ref[...])

    return pl.pallas_call(
        body,
        grid=(n // blk,),
        in_specs=[
            pl.BlockSpec((blk, fin), lambda i: (i, 0)),
            pl.BlockSpec((fin, h), lambda i: (0, 0)),
            pl.BlockSpec((1, h), lambda i: (0, 0)),
        ],
        out_specs=pl.BlockSpec((blk, h), lambda i: (i, 0)),
        out_shape=jax.ShapeDtypeStruct((n, h), jnp.float32),
    )(x, w, b.reshape(1, h))


def _selfmm_chunks(chunks, w, b):
    k = len(chunks)
    n = chunks[0].shape[0]
    h = w.shape[1]
    blk = 1000

    def body(*refs):
        cs = refs[:k]
        w_ref, b_ref, o_ref = refs[k], refs[k + 1], refs[k + 2]
        acc = jnp.broadcast_to(b_ref[...], (blk, h))
        for i in range(k):
            acc = acc + jnp.dot(cs[i][...], w_ref[i * 128:(i + 1) * 128, :],
                                preferred_element_type=jnp.float32)
        o_ref[...] = acc

    return pl.pallas_call(
        body,
        grid=(n // blk,),
        in_specs=[pl.BlockSpec((blk, 128), lambda i: (i, 0))] * k + [
            pl.BlockSpec((k * 128, h), lambda i: (0, 0)),
            pl.BlockSpec((1, h), lambda i: (0, 0)),
        ],
        out_specs=pl.BlockSpec((blk, h), lambda i: (i, 0)),
        out_shape=jax.ShapeDtypeStruct((n, h), jnp.float32),
    )(*chunks, w, b.reshape(1, h))


def _combine(agg_chunks, cnt, s_term, wl):
    k = len(agg_chunks)
    n = agg_chunks[0].shape[0]
    h = wl.shape[1]
    ko = h // 128
    blk = 1000

    def body(*refs):
        aggs = refs[:k]
        cnt_ref, s_ref, w_ref = refs[k], refs[k + 1], refs[k + 2]
        outs = refs[k + 3:]
        acc = jnp.zeros((blk, h), jnp.float32)
        for i in range(k):
            acc = acc + jnp.dot(aggs[i][...], w_ref[i * 128:(i + 1) * 128, :],
                                preferred_element_type=jnp.float32)
        inv = 1.0 / jnp.maximum(cnt_ref[...][:, :1], 1.0)
        hv = jnp.maximum(acc * inv + s_ref[...], 0.0)
        for i in range(ko):
            outs[i][...] = hv[:, i * 128:(i + 1) * 128]

    return pl.pallas_call(
        body,
        grid=(n // blk,),
        in_specs=[pl.BlockSpec((blk, 128), lambda i: (i, 0))] * k + [
            pl.BlockSpec((blk, 128), lambda i: (i, 0)),
            pl.BlockSpec((blk, h), lambda i: (i, 0)),
            pl.BlockSpec((k * 128, h), lambda i: (0, 0)),
        ],
        out_specs=[pl.BlockSpec((blk, 128), lambda i: (i, 0))] * ko,
        out_shape=[jax.ShapeDtypeStruct((n, 128), jnp.float32)] * ko,
    )(*agg_chunks, cnt, s_term, wl)


def _final(agg_chunks, cnt, s_term, w3l, wp, bp, wo, bo):
    k = len(agg_chunks)
    n = agg_chunks[0].shape[0]
    h = w3l.shape[1]
    c = wo.shape[1]
    blk = 1000

    def body(*refs):
        aggs = refs[:k]
        (cnt_ref, s_ref, w3l_ref, wp_ref, bp_ref, wo_ref, bo_ref,
         o_ref) = refs[k:]
        acc = jnp.zeros((blk, h), jnp.float32)
        for i in range(k):
            acc = acc + jnp.dot(aggs[i][...], w3l_ref[i * 128:(i + 1) * 128, :],
                                preferred_element_type=jnp.float32)
        inv = 1.0 / jnp.maximum(cnt_ref[...][:, :1], 1.0)
        h3 = acc * inv + s_ref[...]
        h4 = jnp.maximum(jnp.dot(h3, wp_ref[...],
                                 preferred_element_type=jnp.float32)
                         + bp_ref[...], 0.0)
        o_ref[...] = (jnp.dot(h4, wo_ref[...],
                              preferred_element_type=jnp.float32)
                      + bo_ref[...])

    return pl.pallas_call(
        body,
        grid=(n // blk,),
        in_specs=[pl.BlockSpec((blk, 128), lambda i: (i, 0))] * k + [
            pl.BlockSpec((blk, 128), lambda i: (i, 0)),
            pl.BlockSpec((blk, h), lambda i: (i, 0)),
            pl.BlockSpec((k * 128, h), lambda i: (0, 0)),
            pl.BlockSpec((h, h), lambda i: (0, 0)),
            pl.BlockSpec((1, h), lambda i: (0, 0)),
            pl.BlockSpec((h, c), lambda i: (0, 0)),
            pl.BlockSpec((1, c), lambda i: (0, 0)),
        ],
        out_specs=pl.BlockSpec((blk, c), lambda i: (i, 0)),
        out_shape=jax.ShapeDtypeStruct((n, c), jnp.float32),
    )(*agg_chunks, cnt, s_term, w3l, wp, bp.reshape(1, h), wo,
      bo.reshape(1, c))


def kernel(x, edge_index, W1l, b1l, W1r, b1r, W2l, b2l, W2r, b2r,
           W3l, b3l, W3r, b3r, Wp, bp, Wo, bo):
    n, f_in = x.shape
    e = edge_index.shape[1]
    src = edge_index[0]
    dst = edge_index[1]

    grain = PAIR * NUM_SUBCORES * BATCH
    e_pad = ((e + grain - 1) // grain) * grain
    if e_pad != e:
        src = jnp.concatenate([src, jnp.zeros((e_pad - e,), jnp.int32)])
        dst = jnp.concatenate([dst, jnp.full((e_pad - e,), n, jnp.int32)])

    zeros128 = jnp.zeros((BATCH, 128), jnp.float32)
    ones128 = jnp.ones((BATCH, 128), jnp.float32)

    k_in = f_in // 128
    x_chunks = [x[:, i * 128:(i + 1) * 128] for i in range(k_in)]

    agg1 = _sc_agg(k_in, n, e_pad, True)(
        *x_chunks, src, dst, zeros128, ones128)
    cnt = agg1[-1]
    s1 = _selfmm_dense(x, W1r, b1l + b1r)
    h1 = _combine(list(agg1[:k_in]), cnt, s1, W1l)

    k_h = len(h1)
    sc_h = _sc_agg(k_h, n, e_pad, False)
    agg2 = sc_h(*h1, src, dst, zeros128, ones128)
    s2 = _selfmm_chunks(h1, W2r, b2l + b2r)
    h2 = _combine(list(agg2), cnt, s2, W2l)

    agg3 = sc_h(*h2, src, dst, zeros128, ones128)
    s3 = _selfmm_chunks(h2, W3r, b3l + b3r)
    return _final(list(agg3), cnt, s3, W3l, Wp, bp, Wo, bo)
